# Initial kernel scaffold; baseline (speedup 1.0000x reference)
import jax
import jax.numpy as jnp
from jax.experimental import pallas as pl


def kernel(user_node_id, movie_node_id, movie_x, edge_src_user, edge_dst_movie, label_src_user, label_dst_movie, user_emb, movie_emb, lin_W, lin_b, conv1_rates_Wl, conv1_rates_bl, conv1_rates_Wr, conv1_rev_Wl, conv1_rev_bl, conv1_rev_Wr, conv2_rates_Wl, conv2_rates_bl, conv2_rates_Wr, conv2_rev_Wl, conv2_rev_bl, conv2_rev_Wr):
    raise NotImplementedError("write your pallas kernel here")


# speedup vs baseline: 1.0319x; 1.0319x over previous
import functools

import jax
import jax.numpy as jnp
from jax import lax
from jax.experimental import pallas as pl
from jax.experimental.pallas import tpu as pltpu
from jax.experimental.pallas import tpu_sc as plsc

N_USER = 100000
N_MOVIE = 10000
E = 500000
L = 100000
H = 128

NU_P = 100352
NM_P = 10240
L_P = 102400
E_P = 524288

NC, NS = 2, 16
NW = NC * NS
ER = E_P // 128
ERW = ER // NW

_SC_MESH = dict(mesh=plsc.VectorSubcoreMesh(core_axis_name="c",
                                            subcore_axis_name="s"),
                compiler_params=pltpu.CompilerParams(
                    needs_layout_passes=False))


def _wid():
    return lax.axis_index("s") * NC + lax.axis_index("c")


def _movie_feat_body(x_ref, w_ref, b_ref, emb_ref, o_ref):
    o_ref[...] = (
        jnp.dot(x_ref[...], w_ref[...], preferred_element_type=jnp.float32)
        + b_ref[...] + emb_ref[...]
    )


def _movie_feat(movie_x_pad, lin_W_pad, lin_b, movie_emb_pad):
    blk = 512
    grid = (NM_P // blk,)
    return pl.pallas_call(
        _movie_feat_body,
        grid=grid,
        in_specs=[
            pl.BlockSpec((blk, 32), lambda i: (i, 0)),
            pl.BlockSpec((32, H), lambda i: (0, 0)),
            pl.BlockSpec((1, H), lambda i: (0, 0)),
            pl.BlockSpec((blk, H), lambda i: (i, 0)),
        ],
        out_specs=pl.BlockSpec((blk, H), lambda i: (i, 0)),
        out_shape=jax.ShapeDtypeStruct((NM_P, H), jnp.float32),
    )(movie_x_pad, lin_W_pad, lin_b.reshape(1, H), movie_emb_pad)


def _sage_body(relu, n_s, n_c, *refs):
    s_refs = refs[:n_s]
    c_refs = refs[n_s:n_s + n_c]
    x_ref, wl_ref, wr_ref, b_ref, o_ref = refs[n_s + n_c:]
    ssum = s_refs[0][...]
    for r in s_refs[1:]:
        ssum = ssum + r[...]
    cnt = c_refs[0][...]
    for r in c_refs[1:]:
        cnt = cnt + r[...]
    agg = ssum / jnp.maximum(cnt, 1.0)
    out = (
        jnp.dot(agg, wl_ref[...], preferred_element_type=jnp.float32)
        + jnp.dot(x_ref[...], wr_ref[...], preferred_element_type=jnp.float32)
        + b_ref[...]
    )
    if relu:
        out = jnp.maximum(out, 0.0)
    o_ref[...] = out


def _sage_update(ss, cs, x, Wl, b, Wr, relu):
    n = ss[0].shape[0]
    blk = 512
    grid = (n // blk,)
    in_specs = ([pl.BlockSpec((blk, H), lambda i: (i, 0)) for _ in ss]
                + [pl.BlockSpec((blk, 1), lambda i: (i, 0)) for _ in cs]
                + [
                    pl.BlockSpec((blk, H), lambda i: (i, 0)),
                    pl.BlockSpec((H, H), lambda i: (0, 0)),
                    pl.BlockSpec((H, H), lambda i: (0, 0)),
                    pl.BlockSpec((1, H), lambda i: (0, 0)),
                ])
    return pl.pallas_call(
        functools.partial(_sage_body, relu, len(ss), len(cs)),
        grid=grid,
        in_specs=in_specs,
        out_specs=pl.BlockSpec((blk, H), lambda i: (i, 0)),
        out_shape=jax.ShapeDtypeStruct((n, H), jnp.float32),
    )(*ss, *cs, x, Wl, Wr, b.reshape(1, H))


def _dot_body(a_ref, b_ref, o_ref):
    o_ref[...] = jnp.sum(a_ref[...] * b_ref[...], axis=1, keepdims=True)


def _pair_dot(a, b):
    n = a.shape[0]
    blk = 512
    return pl.pallas_call(
        _dot_body,
        grid=(n // blk,),
        in_specs=[
            pl.BlockSpec((blk, H), lambda i: (i, 0)),
            pl.BlockSpec((blk, H), lambda i: (i, 0)),
        ],
        out_specs=pl.BlockSpec((blk, 1), lambda i: (i, 0)),
        out_shape=jax.ShapeDtypeStruct((n, 1), jnp.float32),
    )(a, b)


def _sc_counts(src2d, dst2d, ones_rows, zc_u, zc_m):
    BR = 16
    NB = ERW // BR
    UCH = NU_P // NS // 16

    @functools.partial(
        pl.kernel,
        out_type=(jax.ShapeDtypeStruct((NC, NU_P, 16), jnp.float32),
                  jax.ShapeDtypeStruct((NC, NM_P, 16), jnp.float32)),
        scratch_types=[
            pltpu.VMEM((128, 16), jnp.float32),
            pltpu.VMEM((BR, 128), jnp.int32),
            pltpu.VMEM((BR, 128), jnp.int32),
            pltpu.VMEM((UCH, 16), jnp.float32),
            pltpu.VMEM_SHARED((NU_P, 16), jnp.float32),
            pltpu.VMEM_SHARED((NM_P, 16), jnp.float32),
            pltpu.SemaphoreType.DMA,
        ],
        **_SC_MESH,
    )
    def k(src_h, dst_h, ones_h, zu_h, zm_h, ocu_h, ocm_h,
          ones_v, sbuf, dbuf, stg, acc_u, acc_m, sem):
        c = lax.axis_index("c")
        s = lax.axis_index("s")
        wid = _wid()
        pltpu.sync_copy(ones_h, ones_v)
        pltpu.sync_copy(zu_h, stg)
        ub = s * (NU_P // NS)
        mb = s * (NM_P // NS)
        for t in range(16):
            pltpu.sync_copy(stg, acc_u.at[pl.ds(ub + t * UCH, UCH)])
        for t in range(2):
            pltpu.sync_copy(stg.at[pl.ds(0, 320)],
                            acc_m.at[pl.ds(mb + t * 320, 320)])
        plsc.subcore_barrier()

        def block(g, _):
            base = wid * ERW + g * BR
            pltpu.sync_copy(src_h.at[pl.ds(base, BR)], sbuf)
            pltpu.sync_copy(dst_h.at[pl.ds(base, BR)], dbuf)
            hs = []
            for i in range(BR):
                hs.append(pltpu.async_copy(ones_v, acc_u.at[sbuf.at[i]],
                                           sem, add=True))
                hs.append(pltpu.async_copy(ones_v, acc_m.at[dbuf.at[i]],
                                           sem, add=True))
            for h in hs:
                h.wait()
            return 0

        lax.fori_loop(0, NB, block, 0)
        plsc.subcore_barrier()
        for t in range(16):
            pltpu.sync_copy(acc_u.at[pl.ds(ub + t * UCH, UCH)], stg)
            pltpu.sync_copy(stg, ocu_h.at[c, pl.ds(ub + t * UCH, UCH)])
        for t in range(2):
            pltpu.sync_copy(acc_m.at[pl.ds(mb + t * 320, 320)],
                            stg.at[pl.ds(0, 320)])
            pltpu.sync_copy(stg.at[pl.ds(0, 320)],
                            ocm_h.at[c, pl.ds(mb + t * 320, 320)])

    return k(src2d, dst2d, ones_rows, zc_u, zc_m)


def _sc_agg_m(u_tab, src2d, dst2d, zrows):
    @functools.partial(
        pl.kernel,
        out_type=jax.ShapeDtypeStruct((NC, NM_P, H), jnp.float32),
        scratch_types=[
            pltpu.VMEM((2, 128), jnp.int32),
            pltpu.VMEM((2, 128), jnp.int32),
            pltpu.VMEM((2, 128, H), jnp.float32),
            pltpu.VMEM((64, H), jnp.float32),
            pltpu.VMEM_SHARED((NM_P, H), jnp.float32),
            pltpu.SemaphoreType.DMA,
            pltpu.SemaphoreType.DMA,
        ],
        **_SC_MESH,
    )
    def k(u_h, src_h, dst_h, z_h, o_h, sidx, didx, rows, stg, acc, gsem, ssem):
        c = lax.axis_index("c")
        s = lax.axis_index("s")
        wid = _wid()
        mb = s * (NM_P // NS)
        pltpu.sync_copy(z_h, stg)
        for t in range(NM_P // NS // 64):
            pltpu.sync_copy(stg, acc.at[pl.ds(mb + t * 64, 64)])
        plsc.subcore_barrier()

        def group(g, _):
            base = wid * ERW + g * 2
            pltpu.sync_copy(src_h.at[pl.ds(base, 2)], sidx)
            pltpu.sync_copy(dst_h.at[pl.ds(base, 2)], didx)
            gh = [pltpu.async_copy(u_h.at[sidx.at[i]], rows.at[i], gsem)
                  for i in range(2)]
            for h in gh:
                h.wait()
            sh = [pltpu.async_copy(rows.at[i], acc.at[didx.at[i]], ssem,
                                   add=True) for i in range(2)]
            for h in sh:
                h.wait()
            return 0

        lax.fori_loop(0, ERW // 2, group, 0)
        plsc.subcore_barrier()
        for t in range(NM_P // NS // 64):
            pltpu.sync_copy(acc.at[pl.ds(mb + t * 64, 64)], stg)
            pltpu.sync_copy(stg, o_h.at[c, pl.ds(mb + t * 64, 64)])

    return k(u_tab, src2d, dst2d, zrows)


CU = 6272
CAP = 5120
FCH = 96
FG = 4 * FCH


def _sc_agg_u(m_tab, src2d, dst2d, zrows):
    @functools.partial(
        pl.kernel,
        out_type=jax.ShapeDtypeStruct((NU_P, H), jnp.float32),
        scratch_types=[
            pltpu.VMEM((16, 128), jnp.int32),
            pltpu.VMEM((16, 128), jnp.int32),
            pltpu.VMEM((CAP,), jnp.int32),
            pltpu.VMEM((CAP,), jnp.int32),
            pltpu.VMEM((4, FCH), jnp.int32),
            pltpu.VMEM((4, FCH), jnp.int32),
            pltpu.VMEM((4, FCH, H), jnp.float32),
            pltpu.VMEM_SHARED((CU + 128, H), jnp.float32),
            pltpu.SemaphoreType.DMA,
            pltpu.SemaphoreType.DMA,
        ],
        **_SC_MESH,
    )
    def k(m_h, src_h, dst_h, z_h, o_h, esrc, edst, packd, packo,
          dstage, ostage, rows, acc, gsem, ssem):
        c = lax.axis_index("c")
        s = lax.axis_index("s")

        nr = NU_P // (NC * CU)

        def rnd(r, _):
            lo = (c * nr + r) * CU
            zb = s * ((CU + 128) // NS)
            pltpu.sync_copy(z_h, rows.at[0])
            for t in range(5):
                pltpu.sync_copy(rows.at[0].at[pl.ds(0, 80)],
                                acc.at[pl.ds(zb + t * 80, 80)])
            plsc.subcore_barrier()

            def sub(b, fill):
                base = s * (ER // NS) + b * 16
                pltpu.sync_copy(src_h.at[pl.ds(base, 16)], esrc)
                pltpu.sync_copy(dst_h.at[pl.ds(base, 16)], edst)

                def vec(t, fill):
                    rr = t // 8
                    j = t % 8
                    u = esrc[rr, pl.ds(j * 16, 16)] - lo
                    m = (u >= 0) & (u < CU)
                    d = edst[rr, pl.ds(j * 16, 16)]
                    mi = jnp.where(m, 1, 0)
                    pos = fill + plsc.cumsum(mi) - 1
                    plsc.store_scatter(packo, [pos], u, mask=m)
                    plsc.store_scatter(packd, [pos], d, mask=m)
                    return fill + jnp.sum(mi)

                return lax.fori_loop(0, 128, vec, fill)

            fill = lax.fori_loop(0, (ER // NS) // 16, sub, 0)

            for t in range(FG // 16):
                packd[pl.ds(fill + t * 16, 16)] = jnp.zeros((16,), jnp.int32)
                packo[pl.ds(fill + t * 16, 16)] = jnp.full((16,), CU, jnp.int32)

            def flushg(g, _):
                for i in range(4):
                    for t in range(FCH // 16):
                        off = g * FG + i * FCH + t * 16
                        dstage[i, pl.ds(t * 16, 16)] = packd[pl.ds(off, 16)]
                        ostage[i, pl.ds(t * 16, 16)] = packo[pl.ds(off, 16)]
                gh = [pltpu.async_copy(m_h.at[dstage.at[i]], rows.at[i], gsem)
                      for i in range(4)]
                for h in gh:
                    h.wait()
                sh = [pltpu.async_copy(rows.at[i], acc.at[ostage.at[i]], ssem,
                                       add=True) for i in range(4)]
                for h in sh:
                    h.wait()
                return 0

            lax.fori_loop(0, (fill + FG - 1) // FG, flushg, 0)
            plsc.subcore_barrier()
            wb = s * (CU // NS)
            for t in range(4):
                pltpu.sync_copy(acc.at[pl.ds(wb + t * 96, 96)], rows.at[0])
                pltpu.sync_copy(rows.at[0],
                                o_h.at[pl.ds(lo + wb + t * 96, 96)])
            pltpu.sync_copy(acc.at[pl.ds(wb + 384, 8)],
                            rows.at[0].at[pl.ds(0, 8)])
            pltpu.sync_copy(rows.at[0].at[pl.ds(0, 8)],
                            o_h.at[pl.ds(lo + wb + 384, 8)])
            plsc.subcore_barrier()
            return 0

        lax.fori_loop(0, 8, rnd, 0)

    return k(m_tab, src2d, dst2d, zrows)


def _sc_label_gather(u_tab, m_tab, ls2d, ld2d):
    CH = L_P // NW // 128

    @functools.partial(
        pl.kernel,
        out_type=(jax.ShapeDtypeStruct((L_P, H), jnp.float32),
                  jax.ShapeDtypeStruct((L_P, H), jnp.float32)),
        scratch_types=[
            pltpu.VMEM((2, 128), jnp.int32),
            pltpu.VMEM((2, 128), jnp.int32),
            pltpu.VMEM((2, 128, H), jnp.float32),
            pltpu.VMEM((2, 128, H), jnp.float32),
            pltpu.SemaphoreType.DMA((2,)),
        ],
        **_SC_MESH,
    )
    def k(u_h, m_h, ls_h, ld_h, ou_h, om_h, idxu, idxm, rowsu, rowsm, sem):
        wid = _wid()
        base = wid * CH

        def stage(j, b):
            pltpu.sync_copy(ls_h.at[base + j], idxu.at[b])
            pltpu.sync_copy(ld_h.at[base + j], idxm.at[b])
            hu = pltpu.async_copy(u_h.at[idxu.at[b]], rowsu.at[b], sem.at[b])
            hm = pltpu.async_copy(m_h.at[idxm.at[b]], rowsm.at[b], sem.at[b])
            return hu, hm

        hs = {0: stage(0, 0)}
        for j in range(CH):
            b = j % 2
            if j + 1 < CH:
                hs[(j + 1) % 2] = stage(j + 1, (j + 1) % 2)
            hu, hm = hs[b]
            hu.wait()
            hm.wait()
            pltpu.sync_copy(rowsu.at[b], ou_h.at[pl.ds((base + j) * 128, 128)])
            pltpu.sync_copy(rowsm.at[b], om_h.at[pl.ds((base + j) * 128, 128)])

    return k(u_tab, m_tab, ls2d, ld2d)


def kernel(user_node_id, movie_node_id, movie_x, edge_src_user, edge_dst_movie,
           label_src_user, label_dst_movie, user_emb, movie_emb, lin_W, lin_b,
           conv1_rates_Wl, conv1_rates_bl, conv1_rates_Wr,
           conv1_rev_Wl, conv1_rev_bl, conv1_rev_Wr,
           conv2_rates_Wl, conv2_rates_bl, conv2_rates_Wr,
           conv2_rev_Wl, conv2_rev_bl, conv2_rev_Wr):
    movie_x_pad = jnp.zeros((NM_P, 32), jnp.float32).at[:N_MOVIE, :20].set(movie_x)
    lin_W_pad = jnp.zeros((32, H), jnp.float32).at[:20].set(lin_W)
    movie_emb_pad = jnp.zeros((NM_P, H), jnp.float32).at[:N_MOVIE].set(movie_emb)
    user_emb_pad = jnp.zeros((NU_P, H), jnp.float32).at[:N_USER].set(user_emb)

    x_user0 = user_emb_pad
    x_movie0 = _movie_feat(movie_x_pad, lin_W_pad, lin_b, movie_emb_pad)

    src2d = (jnp.full((E_P,), N_USER, jnp.int32)
             .at[:E].set(edge_src_user).reshape(ER, 128))
    dst2d = (jnp.full((E_P,), N_MOVIE, jnp.int32)
             .at[:E].set(edge_dst_movie).reshape(ER, 128))

    ones_e = jnp.ones((E, 1), jnp.float32)
    c_u0 = jnp.zeros((NU_P, 1), jnp.float32).at[:N_USER].set(
        jax.ops.segment_sum(ones_e, edge_src_user, num_segments=N_USER))
    c_m0 = jnp.zeros((NM_P, 1), jnp.float32).at[:N_MOVIE].set(
        jax.ops.segment_sum(ones_e, edge_dst_movie, num_segments=N_MOVIE))
    c_u1 = jnp.zeros((NU_P, 1), jnp.float32)
    c_m1 = jnp.zeros((NM_P, 1), jnp.float32)

    def agg_layer(xu, xm, dep):
        pm0 = jnp.zeros((NM_P, H), jnp.float32).at[:N_MOVIE].set(
            jax.ops.segment_sum(jnp.take(xu[:N_USER], edge_src_user, axis=0),
                                edge_dst_movie, num_segments=N_MOVIE))
        pm = [pm0, jnp.zeros((NM_P, H), jnp.float32)]
        s_u = jnp.zeros((NU_P, H), jnp.float32).at[:N_USER].set(
            jax.ops.segment_sum(jnp.take(xm[:N_MOVIE], edge_dst_movie, axis=0),
                                edge_src_user, num_segments=N_USER))
        return pm, [s_u]

    s_m, s_u = agg_layer(x_user0, x_movie0, None)
    x_movie1 = _sage_update(s_m, [c_m0, c_m1], x_movie0, conv1_rates_Wl,
                            conv1_rates_bl, conv1_rates_Wr, relu=True)
    x_user1 = _sage_update(s_u, [c_u0, c_u1], x_user0, conv1_rev_Wl,
                           conv1_rev_bl, conv1_rev_Wr, relu=True)

    s_m, s_u = agg_layer(x_user1, x_movie1, None)
    x_movie2 = _sage_update(s_m, [c_m0, c_m1], x_movie1, conv2_rates_Wl,
                            conv2_rates_bl, conv2_rates_Wr, relu=False)
    x_user2 = _sage_update(s_u, [c_u0, c_u1], x_user1, conv2_rev_Wl,
                           conv2_rev_bl, conv2_rev_Wr, relu=False)

    ls2d = (jnp.zeros((L_P,), jnp.int32)
            .at[:L].set(label_src_user).reshape(L_P // 128, 128))
    ld2d = (jnp.zeros((L_P,), jnp.int32)
            .at[:L].set(label_dst_movie).reshape(L_P // 128, 128))
    ef_u, ef_m = _sc_label_gather(x_user2, x_movie2, ls2d, ld2d)
    out = _pair_dot(ef_u, ef_m)
    return out[:L, 0]
